# Initial kernel scaffold; baseline (speedup 1.0000x reference)
import jax
import jax.numpy as jnp
from jax.experimental import pallas as pl


def kernel(inputs, edge_index, W, b):
    raise NotImplementedError("write your pallas kernel here")


# speedup vs baseline: 7.8219x; 7.8219x over previous
import dataclasses
import functools

import jax
import jax.numpy as jnp
from jax import lax
from jax.experimental import pallas as pl
from jax.experimental.pallas import tpu as pltpu
from jax.experimental.pallas import tpu_sc as plsc

N = 10000
E = 320000
F = 128
NC, NS, L = 2, 16, 16
NW = NC * NS
EPW = E // NW
CAP = EPW + L
NROWS = F // L

_mesh = plsc.VectorSubcoreMesh(core_axis_name="c", subcore_axis_name="s")

_sc_params = pltpu.CompilerParams()
if "needs_layout_passes" in pltpu.CompilerParams.__dataclass_fields__:
    _sc_params = dataclasses.replace(_sc_params, needs_layout_passes=False)


def _wid():
    return lax.axis_index("s") * NC + lax.axis_index("c")


@functools.partial(
    pl.kernel,
    out_type=(
        jax.ShapeDtypeStruct((NW, 14 * F), jnp.float32),
        jax.ShapeDtypeStruct((NW, L), jnp.float32),
        jax.ShapeDtypeStruct((NW, CAP), jnp.int32),
        jax.ShapeDtypeStruct((NW, CAP), jnp.int32),
        jax.ShapeDtypeStruct((NW, L), jnp.int32),
    ),
    mesh=_mesh,
    scratch_types=[
        pltpu.VMEM((EPW + L,), jnp.int32),
        pltpu.VMEM((EPW,), jnp.int32),
        pltpu.VMEM((CAP,), jnp.int32),
        pltpu.VMEM((CAP,), jnp.int32),
        pltpu.VMEM((CAP,), jnp.int32),
        pltpu.VMEM((14 * F,), jnp.float32),
        pltpu.VMEM((L,), jnp.float32),
        pltpu.VMEM((L,), jnp.int32),
        pltpu.VMEM((F,), jnp.float32),
    ],
    compiler_params=_sc_params,
)
def _k1(dst_hbm, src_hbm, x_hbm, psum_hbm, pcnt_hbm, mdst_hbm, msrc_hbm,
        mcnt_hbm, dstb, srcb, dmatch, mdst, msrc, acc, cnt, cntv, rowb):
    w = _wid()
    base = w * EPW
    pltpu.sync_copy(dst_hbm.at[pl.ds(base, EPW)], dstb.at[pl.ds(0, EPW)])
    pltpu.sync_copy(src_hbm.at[pl.ds(base, EPW)], srcb)

    zero16 = jnp.zeros((L,), jnp.float32)

    @pl.loop(0, 14 * F, step=L)
    def _(o):
        acc[pl.ds(o, L)] = zero16

    cnt[...] = zero16
    iota = lax.broadcasted_iota(jnp.int32, (L,), 0)

    def scan_body(i, carry):
        nsrc, ndst = carry
        d = dstb[pl.ds(i * L, L)]
        s = srcb[pl.ds(i * L, L)]
        mask_d = d < 14
        mask_s = s < 14
        eidx = i * L + iota
        plsc.store_compressed(dmatch.at[pl.ds(ndst, L)], eidx, mask=mask_d)
        plsc.store_compressed(mdst.at[pl.ds(nsrc, L)], d, mask=mask_s)
        plsc.store_compressed(msrc.at[pl.ds(nsrc, L)], s, mask=mask_s)
        ndst = ndst + jnp.sum(mask_d.astype(jnp.int32))
        nsrc = nsrc + jnp.sum(mask_s.astype(jnp.int32))
        return nsrc, ndst

    nsrc, ndst = lax.fori_loop(0, EPW // L, scan_body, (0, 0))

    def acc_body(j, _):
        e = dmatch[pl.ds(j, L)][0]
        dv = dstb[pl.ds(e, L)][0]
        pltpu.sync_copy(x_hbm.at[pl.ds((base + e) * F, F)], rowb)
        cnt[...] = cnt[...] + (iota == dv).astype(jnp.float32)
        for c in range(NROWS):
            sl = pl.ds(dv * F + c * L, L)
            acc[sl] = acc[sl] + rowb[pl.ds(c * L, L)]
        return 0

    lax.fori_loop(0, ndst, acc_body, 0)

    cntv[...] = jnp.full((L,), nsrc, jnp.int32)
    pltpu.sync_copy(acc, psum_hbm.at[w])
    pltpu.sync_copy(cnt, pcnt_hbm.at[w])
    pltpu.sync_copy(mdst, mdst_hbm.at[w])
    pltpu.sync_copy(msrc, msrc_hbm.at[w])
    pltpu.sync_copy(cntv, mcnt_hbm.at[w])


def _k2_body(psum_ref, pcnt_ref, w_ref, b_ref, ho_ref, bh_ref):
    sums = jnp.sum(psum_ref[...].reshape(NW, 14, F), axis=0)
    cnts = jnp.sum(pcnt_ref[...], axis=0)[:14]
    node_h = sums / jnp.maximum(cnts, 1.0)[:, None]
    ho = lax.dot_general(node_h, w_ref[...], (((1,), (1,)), ((), ())),
                         preferred_element_type=jnp.float32)
    ho_ref[...] = 0.5 * ho
    bh_ref[...] = 0.5 * b_ref[...]


_k2 = pl.pallas_call(
    _k2_body,
    out_shape=(
        jax.ShapeDtypeStruct((14, F), jnp.float32),
        jax.ShapeDtypeStruct((F,), jnp.float32),
    ),
)


BROWS = 125
FILL_PER_TILE = N // NS
HALF = N // NC
EXP_ROWS = 312

@functools.partial(
    pl.kernel,
    out_type=jax.ShapeDtypeStruct((N, F), jnp.float32),
    mesh=_mesh,
    scratch_types=[
        pltpu.VMEM((14 * F,), jnp.float32),
        pltpu.VMEM((F,), jnp.float32),
        pltpu.VMEM((BROWS, F), jnp.float32),
        pltpu.VMEM((CAP,), jnp.int32),
        pltpu.VMEM((CAP,), jnp.int32),
        pltpu.VMEM((L,), jnp.int32),
        pltpu.VMEM((L, F), jnp.float32),
        pltpu.VMEM((L,), jnp.int32),
        pltpu.VMEM_SHARED((N + 1, F), jnp.float32),
    ],
    compiler_params=_sc_params,
)
def _k3(ho_hbm, bh_hbm, mdst_hbm, msrc_hbm, mcnt_hbm, no3_hbm,
        hob, bhb, fillb, mdstb, msrcb, cntb, rowb, idxb, no3_sp):
    c = lax.axis_index("c")
    s = lax.axis_index("s")
    pltpu.sync_copy(ho_hbm, hob)
    pltpu.sync_copy(bh_hbm, bhb)

    @pl.loop(0, BROWS)
    def _(r):
        for cc in range(NROWS):
            fillb[r, pl.ds(cc * L, L)] = bhb[pl.ds(cc * L, L)]

    for k in range(FILL_PER_TILE // BROWS):
        pltpu.sync_copy(fillb, no3_sp.at[pl.ds(s * FILL_PER_TILE + k * BROWS,
                                               BROWS)])
    plsc.subcore_barrier()

    iota = lax.broadcasted_iota(jnp.int32, (L,), 0)
    for li in range(2):
        lst = s * 2 + li
        pltpu.sync_copy(mcnt_hbm.at[lst], cntb)
        pltpu.sync_copy(mdst_hbm.at[lst], mdstb)
        pltpu.sync_copy(msrc_hbm.at[lst], msrcb)
        n = cntb[...][0]

        def add_body(g, _):
            dv = mdstb[pl.ds(g * L, L)]
            sv = msrcb[pl.ds(g * L, L)]
            valid = (g * L + iota) < n
            dvm = jnp.where(valid, dv, N)
            svm = jnp.where(valid, sv, 0)
            for lane in range(L):
                svl = svm[lane]
                for cc in range(NROWS):
                    rowb[lane, pl.ds(cc * L, L)] = hob[pl.ds(svl * F + cc * L,
                                                             L)]
            idxb[...] = dvm
            pltpu.sync_copy(rowb, no3_sp.at[idxb], add=True)
            return 0

        lax.fori_loop(0, (n + L - 1) // L, add_body, 0)

    plsc.subcore_barrier()

    r0 = pl.multiple_of(c * HALF + s * EXP_ROWS, 8)
    pltpu.sync_copy(no3_sp.at[pl.ds(r0, EXP_ROWS)],
                    no3_hbm.at[pl.ds(r0, EXP_ROWS)])

    @pl.when(s == 0)
    def _():
        r1 = pl.multiple_of(c * HALF + NS * EXP_ROWS, 8)
        pltpu.sync_copy(no3_sp.at[pl.ds(r1, HALF - NS * EXP_ROWS)],
                        no3_hbm.at[pl.ds(r1, HALF - NS * EXP_ROWS)])


R = 80
NCHUNK = EPW // R

@functools.partial(
    pl.kernel,
    out_type=jax.ShapeDtypeStruct((E, F), jnp.float32),
    mesh=_mesh,
    scratch_types=[
        pltpu.VMEM((R,), jnp.int32),
        pltpu.VMEM((R,), jnp.int32),
        pltpu.VMEM((R, F), jnp.float32),
        pltpu.VMEM((R, F), jnp.float32),
        pltpu.SemaphoreType.DMA,
        pltpu.SemaphoreType.DMA,
    ],
    compiler_params=_sc_params,
)
def _k4(src_hbm, dst_hbm, no3_hbm, out_hbm, sidx, didx, gsrc, gdst,
        sem1, sem2):
    w = _wid()
    base = w * EPW

    @pl.loop(0, NCHUNK)
    def _(k):
        gb = base + k * R
        pltpu.sync_copy(src_hbm.at[pl.ds(gb, R)], sidx)
        pltpu.sync_copy(dst_hbm.at[pl.ds(gb, R)], didx)
        cp1 = pltpu.async_copy(no3_hbm.at[sidx], gsrc, sem1)
        cp2 = pltpu.async_copy(no3_hbm.at[didx], gdst, sem2)
        cp1.wait()
        cp2.wait()

        @pl.loop(0, R)
        def _(r):
            for cc in range(NROWS):
                sl = pl.ds(cc * L, L)
                gsrc[r, sl] = gsrc[r, sl] + gdst[r, sl]

        pltpu.sync_copy(gsrc, out_hbm.at[pl.ds(gb, R)])


def kernel(inputs, edge_index, W, b):
    src = edge_index[0]
    dst = edge_index[1]
    x_flat = inputs.reshape(E * F)
    psum, pcnt, mdst, msrc, mcnt = _k1(dst, src, x_flat)
    ho, bh = _k2(psum, pcnt, W, b)
    no3 = _k3(ho.reshape(14 * F), bh, mdst, msrc, mcnt)
    return _k4(src, dst, no3)
